# baseline (speedup 1.0000x reference)
import functools

import jax
import jax.numpy as jnp
from jax import lax
from jax.experimental import pallas as pl
from jax.experimental.pallas import tpu as pltpu
from jax.experimental.pallas import tpu_sc as plsc

_Bb, _N, _D, _H, _E = 2, 10000, 128, 256, 10
_M = _Bb * _N
_NW = 32
_WB = 640
_M2 = _NW * _WB
_T = 512
_A = 2 * _M2
_MAXP = _A + _E * _T
_NT = _MAXP // _T
_TEP = 96
_DUMW = (_MAXP - _A) // _NW
_GR = _M2 // 128
_RPW = _MAXP // _NW


def _router_tc(x_ref, Wr1_ref, br1_ref, Wr2_ref, br2_ref,
               e1_ref, e2_ref, lr1_ref, lr2_ref, g1_ref, g2_ref, hist_ref):
    x = x_ref[...]
    h = jnp.maximum(
        jnp.dot(x, Wr1_ref[...], preferred_element_type=jnp.float32)
        + br1_ref[...][None, :], 0.0)
    logits = (jnp.dot(h, Wr2_ref[...], preferred_element_type=jnp.float32)
              + br2_ref[...][None, :])
    ids = lax.broadcasted_iota(jnp.int32, logits.shape, 1)
    l1 = jnp.max(logits, axis=-1, keepdims=True)
    a1 = jnp.min(jnp.where(logits == l1, ids, _E), axis=-1, keepdims=True)
    a1 = jnp.minimum(a1, _E - 1)
    masked = jnp.where(ids == a1, -jnp.inf, logits)
    l2 = jnp.max(masked, axis=-1, keepdims=True)
    a2 = jnp.min(jnp.where(masked == l2, ids, _E), axis=-1, keepdims=True)
    a2 = jnp.minimum(a2, _E - 1)
    ed = jnp.exp(l2 - l1)
    g1 = 1.0 / (1.0 + ed)
    g2 = ed * g1

    ids16 = lax.broadcasted_iota(jnp.int32, (_WB, 16), 1)
    oh1 = (ids16 == a1).astype(jnp.float32)
    oh2 = (ids16 == a2).astype(jnp.float32)
    ri = lax.broadcasted_iota(jnp.int32, (_WB, _WB), 0)
    ci = lax.broadcasted_iota(jnp.int32, (_WB, _WB), 1)
    slt = (ri > ci).astype(jnp.float32)
    hp = lax.Precision.HIGHEST
    pre1 = jnp.dot(slt, oh1, preferred_element_type=jnp.float32, precision=hp)
    pre2 = jnp.dot(slt, oh2, preferred_element_type=jnp.float32, precision=hp)
    c1 = jnp.sum(oh1, axis=0, keepdims=True)
    c2 = jnp.sum(oh2, axis=0, keepdims=True)
    r1 = jnp.sum(pre1 * oh1, axis=1, keepdims=True)
    r2 = jnp.sum((pre2 + c1) * oh2, axis=1, keepdims=True)

    e1_ref[...] = a1
    e2_ref[...] = a2
    lr1_ref[...] = r1.astype(jnp.int32)
    lr2_ref[...] = r2.astype(jnp.int32)
    g1_ref[...] = g1
    g2_ref[...] = g2
    hist_ref[...] = (c1 + c2).astype(jnp.int32)[None]


def _run_router(xf, Wr1, br1, Wr2, br2):
    full = lambda shape: pl.BlockSpec(shape, lambda i: (0,) * len(shape))
    col = jax.ShapeDtypeStruct((_M2, 1), jnp.int32)
    colf = jax.ShapeDtypeStruct((_M2, 1), jnp.float32)
    return pl.pallas_call(
        _router_tc,
        grid=(_NW,),
        in_specs=[
            pl.BlockSpec((_WB, _D), lambda i: (i, 0)),
            full((_D, 128)), full((128,)), full((128, _E)), full((_E,)),
        ],
        out_specs=[pl.BlockSpec((_WB, 1), lambda i: (i, 0))] * 6
        + [pl.BlockSpec((1, 1, 16), lambda i: (i, 0, 0))],
        out_shape=[col, col, col, col, colf, colf,
                   jax.ShapeDtypeStruct((_NW, 1, 16), jnp.int32)],
    )(xf, Wr1, br1, Wr2, br2)


def _offsets_tc(hist_ref, base_ref, aux_ref):
    hist = hist_ref[...].reshape(_NW, 16).astype(jnp.float32)
    totals = jnp.sum(hist, axis=0, keepdims=True)
    padded = jnp.floor(totals / _T - 1.0 / (2 * _T)) * _T + _T
    li = lax.broadcasted_iota(jnp.int32, (16, 16), 0)
    lj = lax.broadcasted_iota(jnp.int32, (16, 16), 1)
    lte = (li <= lj).astype(jnp.float32)
    hp = lax.Precision.HIGHEST
    incl = jnp.dot(padded, lte, preferred_element_type=jnp.float32, precision=hp)
    offsets = incl - padded
    ri = lax.broadcasted_iota(jnp.int32, (_NW, _NW), 0)
    ci = lax.broadcasted_iota(jnp.int32, (_NW, _NW), 1)
    slt = (ri > ci).astype(jnp.float32)
    pfx = jnp.dot(slt, hist, preferred_element_type=jnp.float32, precision=hp)
    base_ref[...] = (offsets + pfx).astype(jnp.int32)
    st = offsets + totals
    pads = padded - totals
    pinc = jnp.dot(pads, lte, preferred_element_type=jnp.float32, precision=hp)
    pp = pinc - pads
    aux = jnp.concatenate([incl, st, pinc, pp,
                           totals, padded, offsets, offsets], axis=0)
    aux_ref[...] = aux.astype(jnp.int32)


def _run_offsets(hist):
    full = lambda shape: pl.BlockSpec(shape, lambda: (0,) * len(shape))
    return pl.pallas_call(
        _offsets_tc,
        in_specs=[full((_NW, 1, 16))],
        out_specs=[full((_NW, 16)), full((8, 16))],
        out_shape=[jax.ShapeDtypeStruct((_NW, 16), jnp.int32),
                   jax.ShapeDtypeStruct((8, 16), jnp.int32)],
    )(hist)


def _expert_tc(te_ref, xg_ref, W1_ref, b1_ref, W2_ref, b2_ref,
               W3_ref, b3_ref, gsl_ref, og_ref):
    xg = xg_ref[...]
    h1 = jnp.maximum(
        jnp.dot(xg, W1_ref[0], preferred_element_type=jnp.float32)
        + b1_ref[0], 0.0)
    h2 = jnp.maximum(
        jnp.dot(h1, W2_ref[0], preferred_element_type=jnp.float32)
        + b2_ref[0], 0.0)
    o = (jnp.dot(h2, W3_ref[0], preferred_element_type=jnp.float32)
         + b3_ref[0])
    og_ref[...] = o * gsl_ref[...]


def _run_experts(te, xg, W1, b1, W2, b2, W3, b3, gsl2d):
    grid_spec = pltpu.PrefetchScalarGridSpec(
        num_scalar_prefetch=1,
        grid=(_NT,),
        in_specs=[
            pl.BlockSpec((_T, _D), lambda i, te: (i, 0)),
            pl.BlockSpec((1, _D, _H), lambda i, te: (te[i], 0, 0)),
            pl.BlockSpec((1, 1, _H), lambda i, te: (te[i], 0, 0)),
            pl.BlockSpec((1, _H, _H), lambda i, te: (te[i], 0, 0)),
            pl.BlockSpec((1, 1, _H), lambda i, te: (te[i], 0, 0)),
            pl.BlockSpec((1, _H, _D), lambda i, te: (te[i], 0, 0)),
            pl.BlockSpec((1, 1, _D), lambda i, te: (te[i], 0, 0)),
            pl.BlockSpec((_T, 1), lambda i, te: (i, 0)),
        ],
        out_specs=pl.BlockSpec((_T, _D), lambda i, te: (i, 0)),
    )
    return pl.pallas_call(
        _expert_tc,
        grid_spec=grid_spec,
        out_shape=jax.ShapeDtypeStruct((_MAXP, _D), jnp.float32),
    )(te, xg, W1, b1.reshape(_E, 1, _H), W2, b2.reshape(_E, 1, _H),
      W3, b3.reshape(_E, 1, _D), gsl2d)


def _dest_tc(bt_ref, e1_ref, e2_ref, lr1_ref, lr2_ref, d0_ref, d1_ref):
    base = bt_ref[0].astype(jnp.float32)
    ids16 = lax.broadcasted_iota(jnp.int32, (_WB, 16), 1)
    oh1 = (ids16 == e1_ref[...]).astype(jnp.float32)
    oh2 = (ids16 == e2_ref[...]).astype(jnp.float32)
    hp = lax.Precision.HIGHEST
    b0 = jnp.sum(oh1 * base, axis=1, keepdims=True)
    b1 = jnp.sum(oh2 * base, axis=1, keepdims=True)
    d0_ref[...] = b0.astype(jnp.int32) + lr1_ref[...]
    d1_ref[...] = b1.astype(jnp.int32) + lr2_ref[...]


def _run_dest(bt, e1, e2, lr1, lr2):
    col = jax.ShapeDtypeStruct((_M2, 1), jnp.int32)
    blk = pl.BlockSpec((_WB, 1), lambda i: (i, 0))
    return pl.pallas_call(
        _dest_tc,
        grid=(_NW,),
        in_specs=[pl.BlockSpec((1, 1, 16), lambda i: (i, 0, 0)),
                  blk, blk, blk, blk],
        out_specs=[blk, blk],
        out_shape=[col, col],
    )(bt, e1, e2, lr1, lr2)


def _take16(v, idx):
    dnums = lax.GatherDimensionNumbers(
        offset_dims=(), collapsed_slice_dims=(0,), start_index_map=(0,))
    return lax.gather(v, idx[:, None], dnums, (1,),
                      mode=lax.GatherScatterMode.PROMISE_IN_BOUNDS)


def _sc_mesh():
    return plsc.VectorSubcoreMesh(core_axis_name="c", subcore_axis_name="s")


def _dispatch_sc(aux_hbm, d0_hbm, d1_hbm, g1_hbm, g2_hbm,
                 src_hbm, gsl_hbm, te_hbm,
                 aux_v, d0f_v, d1f_v, tok_v, g1_v, g2_v,
                 dum_v, zi_v, zf_v, te_v, sem):
    w = lax.axis_index("s") * 2 + lax.axis_index("c")
    pltpu.sync_copy(aux_hbm, aux_v)
    incl = aux_v[pl.ds(0, 16)]
    st = aux_v[pl.ds(16, 16)]
    pinc = aux_v[pl.ds(32, 16)]
    pp = aux_v[pl.ds(48, 16)]

    iota = lax.iota(jnp.int32, 16)
    off640 = pl.multiple_of(w * _WB, 8)
    cps0 = [pltpu.async_copy(d0_hbm.at[pl.ds(off640, _WB)], d0f_v, sem),
            pltpu.async_copy(d1_hbm.at[pl.ds(off640, _WB)], d1f_v, sem),
            pltpu.async_copy(g1_hbm.at[pl.ds(off640, _WB)], g1_v, sem),
            pltpu.async_copy(g2_hbm.at[pl.ds(off640, _WB)], g2_v, sem)]

    def tok_body(i, c):
        o = pl.multiple_of(i * 16, 16)
        tok_v[pl.ds(o, 16)] = iota + (w * _WB + i * 16)
        return c
    lax.fori_loop(0, _WB // 16, tok_body, 0)

    sum_padded = incl[_E - 1]
    p_last = pinc[_E - 1]
    stl = [st[ex] for ex in range(_E)]
    pincl = [pinc[ex] for ex in range(_E)]
    ppl = [pp[ex] for ex in range(_E)]

    def dum_body(j, c):
        local = iota + j * 16
        d = local + w * _DUMW
        slot = sum_padded + (d - p_last)
        for ex in range(_E - 1, -1, -1):
            slot = jnp.where(d < pincl[ex], stl[ex] + (d - ppl[ex]), slot)
        slot = jnp.where(local < _DUMW, slot, _MAXP)
        o = pl.multiple_of(j * 16, 16)
        dum_v[pl.ds(o, 16)] = slot
        zi_v[pl.ds(o, 16)] = jnp.zeros((16,), jnp.int32)
        zf_v[pl.ds(o, 16)] = jnp.zeros((16,), jnp.float32)
        return c
    lax.fori_loop(0, 16, dum_body, 0)

    for cp in cps0:
        cp.wait()
    done_probe = True
    cps = []
    for cp in []:
        cp.wait()

    @pl.when(w == 0)
    def _():
        def te_body(j, c):
            t = iota + j * 16
            start = t << 9
            cnt = jnp.zeros((16,), jnp.int32)
            for ex in range(_E):
                cnt = cnt + jnp.where(start >= incl[ex], 1, 0).astype(
                    jnp.int32)
            o = pl.multiple_of(j * 16, 16)
            te_v[pl.ds(o, 16)] = jnp.where(cnt >= _E, 0, cnt)
            return c
        lax.fori_loop(0, _TEP // 16, te_body, 0)
        pltpu.sync_copy(te_v, te_hbm)


def _run_dispatch(auxf, d0f, d1f, g1m, g2m):
    g5 = _WB // 128
    kern = functools.partial(
        pl.kernel,
        out_type=[
            jax.ShapeDtypeStruct((_MAXP + 128,), jnp.int32),
            jax.ShapeDtypeStruct((_MAXP + 128,), jnp.float32),
            jax.ShapeDtypeStruct((_TEP,), jnp.int32),
        ],
        mesh=_sc_mesh(),
        scratch_types=[
            pltpu.VMEM((128,), jnp.int32),
            pltpu.VMEM((_WB,), jnp.int32),
            pltpu.VMEM((_WB,), jnp.int32),
            pltpu.VMEM((_WB,), jnp.int32),
            pltpu.VMEM((_WB,), jnp.float32),
            pltpu.VMEM((_WB,), jnp.float32),
            pltpu.VMEM((256,), jnp.int32),
            pltpu.VMEM((256,), jnp.int32),
            pltpu.VMEM((256,), jnp.float32),
            pltpu.VMEM((_TEP,), jnp.int32),
            pltpu.SemaphoreType.DMA,
        ],
    )(_dispatch_sc)
    return kern(auxf, d0f, d1f, g1m, g2m)


def _gather_sc(src_hbm, x_hbm, xg_hbm, s_v, i_v, buf_v, sem):
    w = lax.axis_index("s") * 2 + lax.axis_index("c")
    base = pl.multiple_of(w * _RPW, 8)
    pltpu.sync_copy(src_hbm.at[pl.ds(base, _RPW)], s_v)

    def clamp_body(j, c):
        o = pl.multiple_of(j * 16, 16)
        v = s_v[pl.ds(o, 16)]
        s_v[pl.ds(o, 16)] = jnp.minimum(jnp.maximum(v, 0), _M - 1)
        return c
    lax.fori_loop(0, _RPW // 16, clamp_body, 0)

    nc = _RPW // 128

    def chunk_body(c, carry):
        o = pl.multiple_of(c * 128, 128)
        pltpu.async_copy(x_hbm.at[s_v.at[pl.ds(o, 128)]], buf_v, sem).wait()
        pltpu.sync_copy(buf_v, xg_hbm.at[pl.ds(base + o, 128), :])
        return carry
    lax.fori_loop(0, nc, chunk_body, 0)

    rem = _RPW - nc * 128
    o = nc * 128
    pltpu.async_copy(
        x_hbm.at[s_v.at[pl.ds(o, rem)]],
        buf_v.at[pl.ds(0, rem), :], sem).wait()
    pltpu.sync_copy(buf_v.at[pl.ds(0, rem), :],
                    xg_hbm.at[pl.ds(base + o, rem), :])


def _run_gather(src, xf):
    kern = functools.partial(
        pl.kernel,
        out_type=jax.ShapeDtypeStruct((_MAXP, _D), jnp.float32),
        mesh=_sc_mesh(),
        scratch_types=[
            pltpu.VMEM((_RPW,), jnp.int32),
            pltpu.VMEM((128,), jnp.int32),
            pltpu.VMEM((128, _D), jnp.float32),
            pltpu.SemaphoreType.DMA,
        ],
    )(_gather_sc)
    return kern(src, xf)


def _combine_sc(d0_hbm, d1_hbm, og_hbm, out_hbm, i0_v, i1_v, buf_v, sem):
    w = lax.axis_index("s") * 2 + lax.axis_index("c")

    def chunk(g, rows):
        offt = pl.multiple_of(g * 128, 8)
        pltpu.sync_copy(d0_hbm.at[pl.ds(offt, 128)], i0_v)
        pltpu.sync_copy(d1_hbm.at[pl.ds(offt, 128)], i1_v)
        pltpu.async_copy(og_hbm.at[i0_v], buf_v, sem).wait()
        pltpu.async_copy(og_hbm.at[i1_v], buf_v, sem, add=True).wait()
        off = pl.multiple_of(g * 128, 8)
        pltpu.sync_copy(buf_v.at[pl.ds(0, rows), :],
                        out_hbm.at[pl.ds(off, rows), :])

    nfull = _M // 128
    rem = _M - nfull * 128
    for i in range(4):
        chunk(i * 32 + w, 128)

    g = 128 + w

    @pl.when(g < nfull)
    def _():
        chunk(g, 128)

    @pl.when(g == nfull)
    def _():
        chunk(g, rem)


def _run_combine(d0, d1, og):
    kern = functools.partial(
        pl.kernel,
        out_type=jax.ShapeDtypeStruct((_M, _D), jnp.float32),
        mesh=_sc_mesh(),
        scratch_types=[
            pltpu.VMEM((128,), jnp.int32),
            pltpu.VMEM((128,), jnp.int32),
            pltpu.VMEM((128, _D), jnp.float32),
            pltpu.SemaphoreType.DMA,
        ],
    )(_combine_sc)
    return kern(d0, d1, og)


def kernel(x, Wr1, br1, Wr2, br2, W1, b1, W2, b2, W3, b3):
    xf = x.reshape(_M, _D)
    e1, e2, lr1, lr2, g1, g2, hist = _run_router(xf, Wr1, br1, Wr2, br2)
    bt, aux = _run_offsets(hist)
    d0, d1 = _run_dest(bt.reshape(_NW, 1, 16), e1, e2, lr1, lr2)
    src, gsl, te = _run_dispatch(
        aux.reshape(128), d0.reshape(_M2), d1.reshape(_M2),
        g1.reshape(_M2), g2.reshape(_M2))
    xg = _run_gather(src, xf)
    og = _run_experts(te, xg, W1, b1, W2, b2, W3, b3,
                      gsl[:_MAXP].reshape(_MAXP, 1))
    out = _run_combine(d0.reshape(_M2), d1.reshape(_M2), og)
    return out.reshape(_Bb, _N, _D)


# speedup vs baseline: 14.0403x; 14.0403x over previous
import jax
import jax.numpy as jnp
from jax import lax
from jax.experimental import pallas as pl
from jax.experimental.pallas import tpu as pltpu

_Bb, _N, _D, _H, _E = 2, 10000, 128, 256, 10
_TM = 2000


def _moe_tile(x_ref, Wr1_ref, br1_ref, Wr2_ref, br2_ref,
              W1_ref, b1_ref, W2_ref, b2_ref, W3_ref, b3_ref, out_ref):
    x = x_ref[...]
    h = jnp.maximum(
        jnp.dot(x, Wr1_ref[...], preferred_element_type=jnp.float32)
        + br1_ref[...][None, :], 0.0)
    logits = (jnp.dot(h, Wr2_ref[...], preferred_element_type=jnp.float32)
              + br2_ref[...][None, :])
    ids = lax.broadcasted_iota(jnp.int32, logits.shape, 1)
    l1 = jnp.max(logits, axis=-1, keepdims=True)
    a1 = jnp.min(jnp.where(logits == l1, ids, _E), axis=-1, keepdims=True)
    masked = jnp.where(ids == a1, -jnp.inf, logits)
    l2 = jnp.max(masked, axis=-1, keepdims=True)
    a2 = jnp.min(jnp.where(masked == l2, ids, _E), axis=-1, keepdims=True)
    ed = jnp.exp(l2 - l1)
    g1 = 1.0 / (1.0 + ed)
    g2 = ed / (1.0 + ed)

    xb = x.astype(jnp.bfloat16)
    acc = jnp.zeros((x.shape[0], _D), dtype=jnp.float32)
    for e in range(_E):
        ge = (jnp.where(a1 == e, g1, 0.0) + jnp.where(a2 == e, g2, 0.0))
        h1 = jnp.maximum(
            jnp.dot(xb, W1_ref[e], preferred_element_type=jnp.float32)
            + b1_ref[e][None, :], 0.0)
        h2 = jnp.maximum(
            jnp.dot(h1.astype(jnp.bfloat16), W2_ref[e],
                    preferred_element_type=jnp.float32)
            + b2_ref[e][None, :], 0.0)
        o = (jnp.dot(h2.astype(jnp.bfloat16), W3_ref[e],
                     preferred_element_type=jnp.float32)
             + b3_ref[e][None, :])
        acc = acc + ge * o
    out_ref[...] = acc


def kernel(x, Wr1, br1, Wr2, br2, W1, b1, W2, b2, W3, b3):
    M = _Bb * _N
    xf = x.reshape(M, _D)
    W1b = W1.astype(jnp.bfloat16)
    W2b = W2.astype(jnp.bfloat16)
    W3b = W3.astype(jnp.bfloat16)
    full = lambda shape: pl.BlockSpec(shape, lambda i: (0,) * len(shape))
    out = pl.pallas_call(
        _moe_tile,
        grid=(M // _TM,),
        in_specs=[
            pl.BlockSpec((_TM, _D), lambda i: (i, 0)),
            full((_D, 128)), full((128,)), full((128, _E)), full((_E,)),
            full((_E, _D, _H)), full((_E, _H)),
            full((_E, _H, _H)), full((_E, _H)),
            full((_E, _H, _D)), full((_E, _D)),
        ],
        out_specs=pl.BlockSpec((_TM, _D), lambda i: (i, 0)),
        out_shape=jax.ShapeDtypeStruct((M, _D), jnp.float32),
    )(xf, Wr1, br1, Wr2, br2, W1b, b1, W2b, b2, W3b, b3)
    return out.reshape(_Bb, _N, _D)


# speedup vs baseline: 14.5845x; 1.0388x over previous
import jax
import jax.numpy as jnp
from jax import lax
from jax.experimental import pallas as pl
from jax.experimental.pallas import tpu as pltpu

_Bb, _N, _D, _H, _E = 2, 10000, 128, 256, 10
_TM = 2000


def _moe_tile(x_ref, Wr1_ref, br1_ref, Wr2_ref, br2_ref,
              W1_ref, b1_ref, W2_ref, b2_ref, W3_ref, b3_ref, out_ref):
    x = x_ref[...]
    h = jnp.maximum(
        jnp.dot(x, Wr1_ref[...], preferred_element_type=jnp.float32)
        + br1_ref[...][None, :], 0.0)
    logits = (jnp.dot(h, Wr2_ref[...], preferred_element_type=jnp.float32)
              + br2_ref[...][None, :])
    ids = lax.broadcasted_iota(jnp.int32, logits.shape, 1)
    l1 = jnp.max(logits, axis=-1, keepdims=True)
    a1 = jnp.min(jnp.where(logits == l1, ids, _E), axis=-1, keepdims=True)
    masked = jnp.where(ids == a1, -jnp.inf, logits)
    l2 = jnp.max(masked, axis=-1, keepdims=True)
    a2 = jnp.min(jnp.where(masked == l2, ids, _E), axis=-1, keepdims=True)
    ed = jnp.exp(l2 - l1)
    g1 = 1.0 / (1.0 + ed)
    g2 = ed / (1.0 + ed)

    acc = jnp.zeros((x.shape[0], _D), dtype=jnp.float32)
    for e in range(_E):
        ge = (jnp.where(a1 == e, g1, 0.0) + jnp.where(a2 == e, g2, 0.0))
        h1 = jnp.maximum(
            jnp.dot(x, W1_ref[e], preferred_element_type=jnp.float32)
            + b1_ref[e][None, :], 0.0)
        h2 = jnp.maximum(
            jnp.dot(h1, W2_ref[e], preferred_element_type=jnp.float32)
            + b2_ref[e][None, :], 0.0)
        o = (jnp.dot(h2, W3_ref[e], preferred_element_type=jnp.float32)
             + b3_ref[e][None, :])
        acc = acc + ge * o
    out_ref[...] = acc


def kernel(x, Wr1, br1, Wr2, br2, W1, b1, W2, b2, W3, b3):
    M = _Bb * _N
    xf = x.reshape(M, _D)
    full = lambda shape: pl.BlockSpec(shape, lambda i: (0,) * len(shape))
    out = pl.pallas_call(
        _moe_tile,
        grid=(M // _TM,),
        in_specs=[
            pl.BlockSpec((_TM, _D), lambda i: (i, 0)),
            full((_D, 128)), full((128,)), full((128, _E)), full((_E,)),
            full((_E, _D, _H)), full((_E, _H)),
            full((_E, _H, _H)), full((_E, _H)),
            full((_E, _H, _D)), full((_E, _D)),
        ],
        out_specs=pl.BlockSpec((_TM, _D), lambda i: (i, 0)),
        out_shape=jax.ShapeDtypeStruct((M, _D), jnp.float32),
    )(xf, Wr1, br1, Wr2, br2, W1, b1, W2, b2, W3, b3)
    return out.reshape(_Bb, _N, _D)
